# baseline (speedup 1.0000x reference)
import functools

import jax
import jax.numpy as jnp
from jax import lax
from jax.experimental import pallas as pl
from jax.experimental.pallas import tpu as pltpu
from jax.experimental.pallas import tpu_sc as plsc

_CH = 128
_KC = 3


@functools.lru_cache(maxsize=None)
def _build(N, D, NC, NS):
    NW = NC * NS
    per_w = N // NW
    n_ch = per_w // _CH
    G = n_ch // _KC
    mesh = plsc.VectorSubcoreMesh(core_axis_name="c", subcore_axis_name="s")

    @functools.partial(
        pl.kernel,
        mesh=mesh,
        out_type=jax.ShapeDtypeStruct((N // _CH, _CH, 128), jnp.float32),
        scratch_types=[
            pltpu.VMEM((n_ch, _CH), jnp.int32),
            pltpu.VMEM((2, _KC, _CH, 128), jnp.float32),
            pltpu.SemaphoreType.DMA,
            pltpu.SemaphoreType.DMA,
        ],
        compiler_params=pltpu.CompilerParams(
            use_tc_tiling_on_sc=False, needs_layout_passes=False
        ),
    )
    def k(x_hbm, table_hbm, out_hbm, idx_v, rows_v, sem_a, sem_b):
        wid = lax.axis_index("s") * NC + lax.axis_index("c")
        base = wid * n_ch
        pltpu.sync_copy(x_hbm.at[wid], idx_v)

        def fire(g, b, sem):
            for kk in range(_KC):
                pltpu.async_copy(
                    table_hbm.at[idx_v.at[g * _KC + kk]], rows_v.at[b, kk], sem
                )

        def drain(g, b, sem):
            for kk in range(_KC):
                pltpu.make_async_copy(
                    table_hbm.at[idx_v.at[g * _KC + kk]], rows_v.at[b, kk], sem
                ).wait()

        fire(0, 0, sem_a)

        def step(g, b, sem):
            @pl.when(g + 1 < G)
            def _():
                fire(g + 1, 1 - b, sem_b if b == 0 else sem_a)

            drain(g, b, sem)
            pltpu.sync_copy(
                rows_v.at[b], out_hbm.at[pl.ds(base + g * _KC, _KC)]
            )

        def body(i, carry):
            step(2 * i, 0, sem_a)
            step(2 * i + 1, 1, sem_b)
            return carry

        lax.fori_loop(0, G // 2, body, 0)

    return k


def kernel(x, table):
    B, S = x.shape
    V, D = table.shape
    N = B * S
    info = plsc.get_sparse_core_info()
    NC, NS = info.num_cores, info.num_subcores
    NW = NC * NS
    grain = NW * _CH * _KC * 2
    Np = ((N + grain - 1) // grain) * grain
    xf = x.reshape(-1).astype(jnp.int32)
    if Np != N:
        xf = jnp.concatenate([xf, jnp.zeros((Np - N,), jnp.int32)])
    xf = xf.reshape(NW, Np // (NW * _CH), _CH)
    tbl2 = jnp.pad(table, ((0, 0), (0, 128 - D)))
    out = _build(Np, D, NC, NS)(xf, tbl2)
    out = out.reshape(Np, 128)
    if Np != N:
        out = out[:N]
    return out[:, :D].reshape(B, S, D)


# speedup vs baseline: 1.9018x; 1.9018x over previous
import functools

import jax
import jax.numpy as jnp
from jax import lax
from jax.experimental import pallas as pl
from jax.experimental.pallas import tpu as pltpu
from jax.experimental.pallas import tpu_sc as plsc

_CH = 128
_KC = 2


@functools.lru_cache(maxsize=None)
def _build(N, D, NC, NS):
    NW = NC * NS
    per_w = N // NW
    n_ch = per_w // _CH
    G = n_ch // _KC
    mesh = plsc.VectorSubcoreMesh(core_axis_name="c", subcore_axis_name="s")

    @functools.partial(
        pl.kernel,
        mesh=mesh,
        out_type=jax.ShapeDtypeStruct((N // _CH, _CH, 128), jnp.float32),
        scratch_types=[
            pltpu.VMEM((n_ch, _CH), jnp.int32),
            pltpu.VMEM((2, _KC, _CH, 128), jnp.float32),
            pltpu.SemaphoreType.DMA,
            pltpu.SemaphoreType.DMA,
        ],
        compiler_params=pltpu.CompilerParams(
            use_tc_tiling_on_sc=False, needs_layout_passes=False
        ),
    )
    def k(x_hbm, table_hbm, out_hbm, idx_v, rows_v, sem_a, sem_b):
        wid = lax.axis_index("s") * NC + lax.axis_index("c")
        base = wid * n_ch
        pltpu.sync_copy(x_hbm.at[wid], idx_v)

        def fire(g, b, sem):
            for kk in range(_KC):
                pltpu.async_copy(
                    table_hbm.at[idx_v.at[g * _KC + kk]], rows_v.at[b, kk], sem
                )

        def drain(g, b, sem):
            for kk in range(_KC):
                pltpu.make_async_copy(
                    table_hbm.at[idx_v.at[g * _KC + kk]], rows_v.at[b, kk], sem
                ).wait()

        fire(0, 0, sem_a)

        def step(g, b, sem):
            @pl.when(g + 1 < G)
            def _():
                fire(g + 1, 1 - b, sem_b if b == 0 else sem_a)

            drain(g, b, sem)
            pltpu.sync_copy(
                rows_v.at[b], out_hbm.at[pl.ds(base + g * _KC, _KC)]
            )

        def body(i, carry):
            step(2 * i, 0, sem_a)
            step(2 * i + 1, 1, sem_b)
            return carry

        lax.fori_loop(0, G // 2, body, 0)

    return k


def kernel(x, table):
    B, S = x.shape
    V, D = table.shape
    N = B * S
    info = plsc.get_sparse_core_info()
    NC, NS = info.num_cores, info.num_subcores
    NW = NC * NS
    grain = NW * _CH * _KC * 2
    Np = ((N + grain - 1) // grain) * grain
    xf = x.reshape(-1).astype(jnp.int32)
    if Np != N:
        xf = jnp.concatenate([xf, jnp.zeros((Np - N,), jnp.int32)])
    xf = xf.reshape(NW, Np // (NW * _CH), _CH)
    tbl2 = jnp.pad(table, ((0, 0), (0, 128 - D)))
    out = _build(Np, D, NC, NS)(xf, tbl2)
    out = out.reshape(Np, 128)
    if Np != N:
        out = out[:N]
    return out[:, :D].reshape(B, S, D)


# speedup vs baseline: 1.9044x; 1.0014x over previous
import functools

import jax
import jax.numpy as jnp
from jax import lax
from jax.experimental import pallas as pl
from jax.experimental.pallas import tpu as pltpu
from jax.experimental.pallas import tpu_sc as plsc

_CH = 128
_KC = 2
_NB = 3


@functools.lru_cache(maxsize=None)
def _build(N, NC, NS):
    NW = NC * NS
    per_w = N // NW
    n_ch = per_w // _CH
    G = n_ch // _KC
    mesh = plsc.VectorSubcoreMesh(core_axis_name="c", subcore_axis_name="s")

    @functools.partial(
        pl.kernel,
        mesh=mesh,
        out_type=jax.ShapeDtypeStruct((N // _CH, _CH, 128), jnp.float32),
        scratch_types=[
            pltpu.VMEM((n_ch, _CH), jnp.int32),
            pltpu.VMEM((_NB, _KC, _CH, 128), jnp.float32),
            [pltpu.SemaphoreType.DMA] * _NB,
            [pltpu.SemaphoreType.DMA] * _NB,
        ],
        compiler_params=pltpu.CompilerParams(
            use_tc_tiling_on_sc=False, needs_layout_passes=False
        ),
    )
    def k(x_hbm, table_hbm, out_hbm, idx_v, rows_v, gsems, wsems):
        wid = lax.axis_index("s") * NC + lax.axis_index("c")
        base = wid * n_ch
        pltpu.sync_copy(x_hbm.at[wid], idx_v)

        def fire(g, b):
            for kk in range(_KC):
                pltpu.async_copy(
                    table_hbm.at[idx_v.at[g * _KC + kk]],
                    rows_v.at[b, kk],
                    gsems[b],
                )

        def drain(g, b):
            for kk in range(_KC):
                pltpu.make_async_copy(
                    table_hbm.at[idx_v.at[g * _KC + kk]],
                    rows_v.at[b, kk],
                    gsems[b],
                ).wait()

        def write(g, b):
            pltpu.async_copy(
                rows_v.at[b], out_hbm.at[pl.ds(base + g * _KC, _KC)], wsems[b]
            )

        def write_wait(g, b):
            pltpu.make_async_copy(
                rows_v.at[b], out_hbm.at[pl.ds(base + g * _KC, _KC)], wsems[b]
            ).wait()

        fire(0, 0)
        fire(1, 1)

        def step(g, ph):
            nb = (ph + 2) % _NB

            @pl.when(g + 2 < G)
            def _():
                @pl.when(g >= 1)
                def _():
                    write_wait(g - 1, nb)

                fire(g + 2, nb)

            drain(g, ph)
            write(g, ph)

        def body(i, carry):
            g = _NB * i
            for ph in range(_NB):
                step(g + ph, ph)
            return carry

        lax.fori_loop(0, G // _NB, body, 0)
        for g in range(G - G % _NB, G):
            drain(g, g % _NB)
            write(g, g % _NB)
        for g in range(G - _NB, G):
            write_wait(g, g % _NB)

    return k


def kernel(x, table):
    B, S = x.shape
    V, D = table.shape
    N = B * S
    info = plsc.get_sparse_core_info()
    NC, NS = info.num_cores, info.num_subcores
    NW = NC * NS
    grain = NW * _CH * _KC
    Np = ((N + grain - 1) // grain) * grain
    xf = x.reshape(-1).astype(jnp.int32)
    if Np != N:
        xf = jnp.concatenate([xf, jnp.zeros((Np - N,), jnp.int32)])
    xf = xf.reshape(NW, Np // (NW * _CH), _CH)
    tbl2 = jnp.pad(table, ((0, 0), (0, 128 - D)))
    out = _build(Np, NC, NS)(xf, tbl2)
    out = out.reshape(Np, 128)
    if Np != N:
        out = out[:N]
    return out[:, :D].reshape(B, S, D)


# speedup vs baseline: 2.2229x; 1.1673x over previous
import functools

import jax
import jax.numpy as jnp
from jax import lax
from jax.experimental import pallas as pl
from jax.experimental.pallas import tpu as pltpu
from jax.experimental.pallas import tpu_sc as plsc

_CH = 128
_KC = 2
_NB = 3


@functools.lru_cache(maxsize=None)
def _build(N, NC, NS):
    NW = NC * NS
    per_w = N // NW
    n_ch = per_w // _CH
    G = n_ch // _KC
    mesh = plsc.VectorSubcoreMesh(core_axis_name="c", subcore_axis_name="s")

    @functools.partial(
        pl.kernel,
        mesh=mesh,
        out_type=jax.ShapeDtypeStruct((N // _CH, _CH, 128), jnp.float32),
        scratch_types=[
            pltpu.VMEM((n_ch, _CH), jnp.int32),
            pltpu.VMEM((_NB, _KC, _CH, 64), jnp.float32),
            [pltpu.SemaphoreType.DMA] * _NB,
            [pltpu.SemaphoreType.DMA] * _NB,
        ],
        compiler_params=pltpu.CompilerParams(
            use_tc_tiling_on_sc=False, needs_layout_passes=False
        ),
    )
    def k(x_hbm, table_hbm, out_hbm, idx_v, rows_v, gsems, wsems):
        wid = lax.axis_index("s") * NC + lax.axis_index("c")
        base = wid * n_ch
        pltpu.sync_copy(x_hbm.at[wid], idx_v)

        def fire(g, b):
            for kk in range(_KC):
                pltpu.async_copy(
                    table_hbm.at[idx_v.at[g * _KC + kk]],
                    rows_v.at[b, kk],
                    gsems[b],
                )

        def drain(g, b):
            for kk in range(_KC):
                pltpu.make_async_copy(
                    table_hbm.at[idx_v.at[g * _KC + kk]],
                    rows_v.at[b, kk],
                    gsems[b],
                ).wait()

        def write(g, b):
            pltpu.async_copy(
                rows_v.at[b],
                out_hbm.at[pl.ds(base + g * _KC, _KC), :, pl.ds(0, 64)],
                wsems[b],
            )

        def write_wait(g, b):
            pltpu.make_async_copy(
                rows_v.at[b],
                out_hbm.at[pl.ds(base + g * _KC, _KC), :, pl.ds(0, 64)],
                wsems[b],
            ).wait()

        fire(0, 0)
        fire(1, 1)

        def step(g, ph):
            nb = (ph + 2) % _NB

            @pl.when(g + 2 < G)
            def _():
                @pl.when(g >= 1)
                def _():
                    write_wait(g - 1, nb)

                fire(g + 2, nb)

            drain(g, ph)
            write(g, ph)

        def body(i, carry):
            g = _NB * i
            for ph in range(_NB):
                step(g + ph, ph)
            return carry

        lax.fori_loop(0, G // _NB, body, 0)
        for g in range(G - G % _NB, G):
            drain(g, g % _NB)
            write(g, g % _NB)
        for g in range(G - _NB, G):
            write_wait(g, g % _NB)

    return k


def kernel(x, table):
    B, S = x.shape
    V, D = table.shape
    N = B * S
    info = plsc.get_sparse_core_info()
    NC, NS = info.num_cores, info.num_subcores
    NW = NC * NS
    grain = NW * _CH * _KC
    Np = ((N + grain - 1) // grain) * grain
    xf = x.reshape(-1).astype(jnp.int32) * 2
    if Np != N:
        xf = jnp.concatenate([xf, jnp.zeros((Np - N,), jnp.int32)])
    xf = xf.reshape(NW, Np // (NW * _CH), _CH)
    tbl2 = jnp.pad(table, ((0, 0), (0, 128 - D))).reshape(2 * V, D)
    out = _build(Np, NC, NS)(xf, tbl2)
    out = out.reshape(Np, 128)
    if Np != N:
        out = out[:N]
    return out[:, :D].reshape(B, S, D)
